# baseline (speedup 1.0000x reference)
import jax
import jax.numpy as jnp
from jax.experimental import pallas as pl
from jax.experimental.pallas import tpu as pltpu

BATCH, TIME, CHANNELS = 32, 2048, 1024
KEEP = 16
SUB = 16
LANE = TIME // SUB


def _body(x_ref, w_ref, o_ref):
    xb = x_ref[...]
    x3 = xb.reshape(SUB, LANE, CHANNELS)
    s_col = jax.lax.dot_general(
        xb, w_ref[...],
        (((1,), (0,)), ((), ())),
        precision=jax.lax.Precision.DEFAULT,
        preferred_element_type=jnp.float32,
    )
    s = s_col.reshape(SUB, LANE)

    gidx = (jax.lax.broadcasted_iota(jnp.int32, (SUB, LANE), 0) * LANE
            + jax.lax.broadcasted_iota(jnp.int32, (SUB, LANE), 1)
            ).astype(jnp.float32)
    big = jnp.float32(TIME)
    neg = jnp.float32(-jnp.inf)
    one = jnp.float32(1.0)
    zero = jnp.float32(0.0)

    colrank = jnp.zeros((SUB, LANE), jnp.float32)
    for r in range(1, SUB):
        sr = jnp.concatenate([s[r:], s[:r]], axis=0)
        gr = jnp.concatenate([gidx[r:], gidx[:r]], axis=0)
        gt = (sr > s) | ((sr == s) & (gr < gidx))
        colrank = colrank + jnp.where(gt, one, zero)

    ptr = jnp.zeros((1, LANE), jnp.float32)
    cand_v = jnp.max(jnp.where(colrank == zero, s, neg), axis=0, keepdims=True)
    cand_g = jnp.min(jnp.where(colrank == zero, gidx, big), axis=0, keepdims=True)
    for _ in range(KEEP):
        m = jnp.max(cand_v, axis=1, keepdims=True)
        g = jnp.min(jnp.where(cand_v == m, cand_g, big),
                    axis=1, keepdims=True)
        ptr = ptr + jnp.where(cand_g == g, one, zero)
        onehot = colrank == ptr
        cand_v = jnp.max(jnp.where(onehot, s, neg), axis=0, keepdims=True)
        cand_g = jnp.min(jnp.where(onehot, gidx, big), axis=0, keepdims=True)

    mask = jnp.minimum(jnp.maximum(ptr - colrank, zero), one)
    o_ref[...] = (x3 * mask[:, :, None]).reshape(TIME, CHANNELS)


def kernel(x, attn_W, attn_b):
    del attn_b
    return pl.pallas_call(
        _body,
        grid=(BATCH,),
        in_specs=[
            pl.BlockSpec((None, TIME, CHANNELS), lambda b: (b, 0, 0)),
            pl.BlockSpec((CHANNELS, 1), lambda b: (0, 0)),
        ],
        out_specs=pl.BlockSpec((None, TIME, CHANNELS), lambda b: (b, 0, 0)),
        out_shape=jax.ShapeDtypeStruct((BATCH, TIME, CHANNELS), x.dtype),
        compiler_params=pltpu.CompilerParams(
            dimension_semantics=("parallel",),
        ),
    )(x, attn_W)
